# baseline (speedup 1.0000x reference)
import functools

import jax
import jax.numpy as jnp
from jax import lax
from jax.experimental import pallas as pl
from jax.experimental.pallas import tpu as pltpu
from jax.experimental.pallas import tpu_sc as plsc

VOCAB_SIZE = 8192
N_ROWS = 8192
NUM_CORES = 2
NUM_SUBCORES = 16
NUM_WORKERS = NUM_CORES * NUM_SUBCORES
ROWS_PER_WORKER = N_ROWS // NUM_WORKERS
LANES = 16
GROUPS = ROWS_PER_WORKER // LANES

TILE_ELEMS = 65536
SHARED_ELEMS = TILE_ELEMS * 4
WORKER_ELEMS = ROWS_PER_WORKER * VOCAB_SIZE
CHUNKS_PER_WORKER = WORKER_ELEMS // SHARED_ELEMS


def _sc_body(ids_hbm, out_hbm, ids_v, zbuf, idx0, idx1, vals, zshared, sem, sem2):
    cid = lax.axis_index("c")
    sid = lax.axis_index("s")
    wid = sid * NUM_CORES + cid
    base_row = wid * ROWS_PER_WORKER

    pltpu.sync_copy(ids_hbm.at[pl.ds(base_row, ROWS_PER_WORKER)], ids_v)

    zeros16 = jnp.zeros((LANES,), jnp.float32)

    def zero_body(k, carry):
        for u in range(8):
            zbuf[pl.ds((k * 8 + u) * LANES, LANES)] = zeros16
        return carry

    lax.fori_loop(0, TILE_ELEMS // (LANES * 8), zero_body, 0)
    pl.when(sid < 4)(lambda: pltpu.sync_copy(zbuf, zshared.at[pl.ds(sid * TILE_ELEMS, TILE_ELEMS)]))

    lane = lax.broadcasted_iota(jnp.int32, (LANES,), 0)
    fives = jnp.full((LANES,), 5.0, jnp.float32)
    for g in range(GROUPS):
        vec = ids_v[pl.ds(g * LANES, LANES)]
        col = lax.rem(vec, VOCAB_SIZE)
        flat = (base_row + g * LANES + lane) * VOCAB_SIZE + col
        if g < GROUPS // 2:
            idx0[pl.ds(g * LANES, LANES)] = flat
            vals[pl.ds(g * LANES, LANES)] = fives
        else:
            idx1[pl.ds((g - GROUPS // 2) * LANES, LANES)] = flat

    plsc.subcore_barrier()

    copies = []
    for c in range(CHUNKS_PER_WORKER):
        start = base_row * VOCAB_SIZE + c * SHARED_ELEMS
        copies.append(
            pltpu.async_copy(zshared, out_hbm.at[pl.ds(start, SHARED_ELEMS)], sem)
        )
    for cp in copies:
        cp.wait()

    pltpu.async_copy(vals, out_hbm.at[idx0], sem2).wait()
    pltpu.async_copy(vals, out_hbm.at[idx1], sem2).wait()


_sc_kernel = functools.partial(
    pl.kernel,
    out_type=jax.ShapeDtypeStruct((N_ROWS * VOCAB_SIZE,), jnp.float32),
    mesh=plsc.VectorSubcoreMesh(core_axis_name="c", subcore_axis_name="s"),
    scratch_types=[
        pltpu.VMEM((ROWS_PER_WORKER,), jnp.int32),
        pltpu.VMEM((TILE_ELEMS,), jnp.float32),
        pltpu.VMEM((ROWS_PER_WORKER // 2,), jnp.int32),
        pltpu.VMEM((ROWS_PER_WORKER // 2,), jnp.int32),
        pltpu.VMEM((ROWS_PER_WORKER // 2,), jnp.float32),
        pltpu.VMEM_SHARED((SHARED_ELEMS,), jnp.float32),
        pltpu.SemaphoreType.DMA,
        pltpu.SemaphoreType.DMA,
    ],
)(_sc_body)


def kernel(input_ids):
    bs, seq = input_ids.shape
    out = _sc_kernel(input_ids.reshape(-1))
    return out.reshape(bs, seq, VOCAB_SIZE)


# speedup vs baseline: 1.1353x; 1.1353x over previous
import functools

import jax
import jax.numpy as jnp
from jax import lax
from jax.experimental import pallas as pl
from jax.experimental.pallas import tpu as pltpu
from jax.experimental.pallas import tpu_sc as plsc

VOCAB_SIZE = 8192
N_ROWS = 8192
NUM_CORES = 2
NUM_SUBCORES = 16
NUM_WORKERS = NUM_CORES * NUM_SUBCORES
ROWS_PER_WORKER = N_ROWS // NUM_WORKERS
LANES = 16
GROUPS = ROWS_PER_WORKER // LANES

TILE_ELEMS = 65536
SHARED_ELEMS = TILE_ELEMS * 4
WORKER_ELEMS = ROWS_PER_WORKER * VOCAB_SIZE
CHUNKS_PER_WORKER = WORKER_ELEMS // SHARED_ELEMS


def _sc_body(ids_hbm, out_hbm, ids_v, zbuf, idx0, idx1, vals, zshared, sem, sem2):
    cid = lax.axis_index("c")
    sid = lax.axis_index("s")
    wid = sid * NUM_CORES + cid
    base_row = wid * ROWS_PER_WORKER

    pltpu.sync_copy(ids_hbm.at[pl.ds(base_row, ROWS_PER_WORKER)], ids_v)

    zeros16 = jnp.zeros((LANES,), jnp.float32)

    def zero_body(k, carry):
        for u in range(8):
            zbuf[pl.ds((k * 8 + u) * LANES, LANES)] = zeros16
        return carry

    lax.fori_loop(0, TILE_ELEMS // (LANES * 8), zero_body, 0)
    pl.when(sid < 4)(lambda: pltpu.sync_copy(zbuf, zshared.at[pl.ds(sid * TILE_ELEMS, TILE_ELEMS)]))

    lane = lax.broadcasted_iota(jnp.int32, (LANES,), 0)
    fives = jnp.full((LANES,), 5.0, jnp.float32)
    for g in range(GROUPS):
        vec = ids_v[pl.ds(g * LANES, LANES)]
        col = lax.rem(vec, VOCAB_SIZE)
        flat = (base_row + g * LANES + lane) * VOCAB_SIZE + col
        if g < GROUPS // 2:
            idx0[pl.ds(g * LANES, LANES)] = flat
            vals[pl.ds(g * LANES, LANES)] = fives
        else:
            idx1[pl.ds((g - GROUPS // 2) * LANES, LANES)] = flat

    plsc.subcore_barrier()

    half = WORKER_ELEMS // 2
    copies = []
    for c in range(half // TILE_ELEMS):
        start = base_row * VOCAB_SIZE + c * TILE_ELEMS
        copies.append(
            pltpu.async_copy(zbuf, out_hbm.at[pl.ds(start, TILE_ELEMS)], sem)
        )
    for c in range(half // SHARED_ELEMS):
        start = base_row * VOCAB_SIZE + half + c * SHARED_ELEMS
        copies.append(
            pltpu.async_copy(zshared, out_hbm.at[pl.ds(start, SHARED_ELEMS)], sem)
        )
    for cp in copies:
        cp.wait()

    pltpu.async_copy(vals, out_hbm.at[idx0], sem2).wait()
    pltpu.async_copy(vals, out_hbm.at[idx1], sem2).wait()


_sc_kernel = functools.partial(
    pl.kernel,
    out_type=jax.ShapeDtypeStruct((N_ROWS * VOCAB_SIZE,), jnp.float32),
    mesh=plsc.VectorSubcoreMesh(core_axis_name="c", subcore_axis_name="s"),
    scratch_types=[
        pltpu.VMEM((ROWS_PER_WORKER,), jnp.int32),
        pltpu.VMEM((TILE_ELEMS,), jnp.float32),
        pltpu.VMEM((ROWS_PER_WORKER // 2,), jnp.int32),
        pltpu.VMEM((ROWS_PER_WORKER // 2,), jnp.int32),
        pltpu.VMEM((ROWS_PER_WORKER // 2,), jnp.float32),
        pltpu.VMEM_SHARED((SHARED_ELEMS,), jnp.float32),
        pltpu.SemaphoreType.DMA,
        pltpu.SemaphoreType.DMA,
    ],
)(_sc_body)


def kernel(input_ids):
    bs, seq = input_ids.shape
    out = _sc_kernel(input_ids.reshape(-1))
    return out.reshape(bs, seq, VOCAB_SIZE)
